# baseline (speedup 1.0000x reference)
import functools

import jax
import jax.numpy as jnp
from jax import lax
from jax.experimental import pallas as pl
from jax.experimental.pallas import tpu as pltpu
from jax.experimental.pallas import tpu_sc as plsc

B = 16384
D = 32
NCTX = 5

_info = plsc.get_sparse_core_info()
NC = _info.num_cores
NS = _info.num_subcores
L = _info.num_lanes
NW = NC * NS
BPW = B // NW
CHUNK = 128
NT_CH = BPW // CHUNK
NCC = BPW * NCTX // CHUNK


def _make_sc_kernel():
    mesh = plsc.VectorSubcoreMesh(core_axis_name="c", subcore_axis_name="s")

    @functools.partial(
        pl.kernel,
        mesh=mesh,
        compiler_params=pltpu.CompilerParams(
            use_tc_tiling_on_sc=False, needs_layout_passes=False),
        out_type=jax.ShapeDtypeStruct((B * NCTX,), jnp.float32),
        scratch_types=[
            pltpu.VMEM((BPW,), jnp.int32),
            pltpu.VMEM((BPW * NCTX,), jnp.int32),
            pltpu.VMEM((BPW, D), jnp.float32),
            pltpu.VMEM((BPW * NCTX, D), jnp.float32),
            pltpu.VMEM((BPW * NCTX,), jnp.float32),
            pltpu.SemaphoreType.DMA,
        ],
    )
    def sc_kernel(tgt_idx_hbm, ctx_idx_hbm, tgt_tab_hbm, ctx_tab_hbm,
                  out_hbm, idx_tv, idx_cv, tgt_rows, ctx_rows, out_v, sem):
        wid = lax.axis_index("s") * NC + lax.axis_index("c")

        pltpu.sync_copy(tgt_idx_hbm.at[pl.ds(wid * BPW, BPW)], idx_tv)
        pltpu.sync_copy(ctx_idx_hbm.at[pl.ds(wid * BPW * NCTX, BPW * NCTX)],
                        idx_cv)

        copies = []
        for j in range(NT_CH):
            copies.append(pltpu.async_copy(
                tgt_tab_hbm.at[idx_tv.at[pl.ds(j * CHUNK, CHUNK)]],
                tgt_rows.at[pl.ds(j * CHUNK, CHUNK)], sem))
        for j in range(NCC):
            copies.append(pltpu.async_copy(
                ctx_tab_hbm.at[idx_cv.at[pl.ds(j * CHUNK, CHUNK)]],
                ctx_rows.at[pl.ds(j * CHUNK, CHUNK)], sem))
        for cp in copies:
            cp.wait()

        lanes = lax.iota(jnp.int32, L)

        def group_body(g, carry):
            bidx = g * L + lanes
            crow = bidx * NCTX
            accs = [jnp.zeros((L,), jnp.float32) for _ in range(NCTX)]
            for e in range(D):
                ecol = jnp.full((L,), e, jnp.int32)
                t_e = plsc.load_gather(tgt_rows, [bidx, ecol])
                for c in range(NCTX):
                    c_e = plsc.load_gather(ctx_rows, [crow + c, ecol])
                    accs[c] = accs[c] + t_e * c_e
            for c in range(NCTX):
                plsc.store_scatter(out_v, [crow + c], accs[c])
            return carry

        lax.fori_loop(0, BPW // L, group_body, 0)

        pltpu.sync_copy(out_v, out_hbm.at[pl.ds(wid * BPW * NCTX, BPW * NCTX)])

    return sc_kernel


_sc_kernel = _make_sc_kernel()


def kernel(target, context, target_table, context_table):
    tgt_idx = target.reshape(B)
    ctx_idx = context.reshape(B * NCTX)
    out = _sc_kernel(tgt_idx, ctx_idx, target_table, context_table)
    return out.reshape(B, NCTX)
